# Initial kernel scaffold; baseline (speedup 1.0000x reference)
import jax
import jax.numpy as jnp
from jax.experimental import pallas as pl


def kernel(sparse, dense, W, b, emb0, emb1, emb2, emb3, emb4, emb5, emb6, emb7, emb8, emb9, emb10, emb11, emb12, emb13, emb14, emb15, emb16, emb17, emb18, emb19):
    raise NotImplementedError("write your pallas kernel here")


# speedup vs baseline: 4.9742x; 4.9742x over previous
import functools

import jax
import jax.numpy as jnp
from jax import lax
from jax.experimental import pallas as pl
from jax.experimental.pallas import tpu as pltpu
from jax.experimental.pallas import tpu_sc as plsc

EMB = 8
N_SP = 20
N_DN = 10
D_OUT = N_SP * EMB + N_DN * EMB
BATCH = 16384
NC = 2
NS = 16
NW = NC * NS
RPW = BATCH // NW
CHUNK = 128
GROUPS = CHUNK // 16


def _body(offs, sp_ref, dn_ref, tb_ref, wb_ref, out_ref,
          sp_v, dn_v, tb_v, wb_v, ob_v):
    cid = lax.axis_index("c")
    sid = lax.axis_index("s")
    wid = sid * NC + cid
    base = wid * RPW
    pltpu.sync_copy(sp_ref.at[pl.ds(base * N_SP, RPW * N_SP)], sp_v)
    pltpu.sync_copy(dn_ref.at[pl.ds(base * N_DN, RPW * N_DN)], dn_v)
    pltpu.sync_copy(tb_ref, tb_v)
    pltpu.sync_copy(wb_ref, wb_v)
    iota = lax.iota(jnp.int32, 16)
    iota_sp = iota * N_SP
    iota_dn = iota * N_DN
    iota_out = iota * D_OUT

    def chunk_body(chunk, carry):
        r0c = chunk * CHUNK

        def group(g, carry):
            r0 = r0c + g * 16
            rows_sp = iota_sp + r0 * N_SP
            rows_dn = iota_dn + r0 * N_DN
            rows_out = iota_out + g * (16 * D_OUT)
            for i in range(N_SP):
                sidx = plsc.load_gather(sp_v, [rows_sp + i])
                t8 = sidx * EMB
                for j in range(EMB):
                    v = plsc.load_gather(tb_v, [t8 + (offs[i] * EMB + j)])
                    plsc.store_scatter(ob_v, [rows_out + (EMB * i + j)], v)
            dcols = [plsc.load_gather(dn_v, [rows_dn + k])
                     for k in range(N_DN)]
            for o in range(N_DN * EMB):
                wrow = wb_v[pl.ds(o * 16, 16)]
                acc = jnp.full((16,), 0.0, jnp.float32) + wrow[N_DN]
                for k in range(N_DN):
                    acc = acc + dcols[k] * wrow[k]
                plsc.store_scatter(ob_v, [rows_out + (N_SP * EMB + o)], acc)
            return carry

        lax.fori_loop(0, GROUPS, group, 0)
        pltpu.sync_copy(
            ob_v, out_ref.at[pl.ds((base + r0c) * D_OUT, CHUNK * D_OUT)])
        return carry

    lax.fori_loop(0, RPW // CHUNK, chunk_body, 0)


def kernel(sparse, dense, W, b, emb0, emb1, emb2, emb3, emb4, emb5, emb6,
           emb7, emb8, emb9, emb10, emb11, emb12, emb13, emb14, emb15,
           emb16, emb17, emb18, emb19):
    embs = [emb0, emb1, emb2, emb3, emb4, emb5, emb6, emb7, emb8, emb9,
            emb10, emb11, emb12, emb13, emb14, emb15, emb16, emb17, emb18,
            emb19]
    offs = []
    t = 0
    for e in embs:
        offs.append(t)
        t += e.shape[0]
    table = jnp.concatenate(embs, axis=0).reshape(-1)
    wb = jnp.zeros((N_DN * EMB, 16), jnp.float32)
    wb = wb.at[:, :N_DN].set(W).at[:, N_DN].set(b).reshape(-1)

    mesh = plsc.VectorSubcoreMesh(core_axis_name="c", subcore_axis_name="s")
    k = pl.kernel(
        functools.partial(_body, tuple(offs)),
        mesh=mesh,
        compiler_params=pltpu.CompilerParams(needs_layout_passes=False),
        out_type=jax.ShapeDtypeStruct((BATCH * D_OUT,), jnp.float32),
        scratch_types=[
            pltpu.VMEM((RPW * N_SP,), jnp.int32),
            pltpu.VMEM((RPW * N_DN,), jnp.float32),
            pltpu.VMEM((t * EMB,), jnp.float32),
            pltpu.VMEM((N_DN * EMB * 16,), jnp.float32),
            pltpu.VMEM((CHUNK * D_OUT,), jnp.float32),
        ],
    )
    out = k(sparse.reshape(-1), dense.reshape(-1), table, wb)
    return out.reshape(BATCH, D_OUT)
